# Initial kernel scaffold; baseline (speedup 1.0000x reference)
import jax
import jax.numpy as jnp
from jax.experimental import pallas as pl


def kernel(features, edge_index, edge_weight, W):
    raise NotImplementedError("write your pallas kernel here")


# speedup vs baseline: 4.4946x; 4.4946x over previous
import functools

import jax
import jax.numpy as jnp
from jax import lax
from jax.experimental import pallas as pl
from jax.experimental.pallas import tpu as pltpu
from jax.experimental.pallas import tpu_sc as plsc

NC = 2
NS = 16
NW = NC * NS
LANES = 16
CHUNK = 80


def _sc_aggregate(features, src, dst, wgt, n_pad):
    n_nodes, d = features.shape
    n_edges = src.shape[0]
    assert n_edges % NW == 0
    e_w = n_edges // NW
    assert e_w % CHUNK == 0
    n_chunks = e_w // CHUNK
    assert n_pad % (NS * 8) == 0
    rows_w = n_pad // NS
    zrows = 128
    assert rows_w % zrows == 0
    d_vecs = d // LANES

    mesh = plsc.VectorSubcoreMesh(core_axis_name="c", subcore_axis_name="s")

    @functools.partial(
        pl.kernel,
        out_type=jax.ShapeDtypeStruct((NC, n_pad, d), jnp.float32),
        mesh=mesh,
        scratch_types=[
            pltpu.VMEM((CHUNK,), jnp.int32),
            pltpu.VMEM((CHUNK,), jnp.int32),
            pltpu.VMEM((CHUNK,), jnp.float32),
            pltpu.VMEM((CHUNK, d), jnp.float32),
            pltpu.VMEM((zrows, d), jnp.float32),
            pltpu.VMEM_SHARED((n_pad, d), jnp.float32),
            pltpu.SemaphoreType.DMA,
        ],
    )
    def agg(feat_hbm, src_hbm, dst_hbm, wgt_hbm, out_hbm,
            src_v, dst_v, w_v, rows_v, zbuf, acc, sem):
        c = lax.axis_index("c")
        s = lax.axis_index("s")
        wid = c * NS + s

        def zfill(i, carry):
            for j in range(d_vecs):
                zbuf[i, pl.ds(j * LANES, LANES)] = jnp.zeros((LANES,), jnp.float32)
            return carry
        lax.fori_loop(0, zrows, zfill, 0)
        r0 = s * rows_w

        def zcopy(i, carry):
            pltpu.sync_copy(zbuf, acc.at[pl.ds(r0 + i * zrows, zrows), :])
            return carry
        lax.fori_loop(0, rows_w // zrows, zcopy, 0)
        plsc.subcore_barrier()

        e0 = wid * e_w

        def chunk_body(i, carry):
            base = e0 + i * CHUNK
            pltpu.sync_copy(src_hbm.at[pl.ds(base, CHUNK)], src_v)
            pltpu.sync_copy(dst_hbm.at[pl.ds(base, CHUNK)], dst_v)
            pltpu.sync_copy(wgt_hbm.at[pl.ds(base, CHUNK)], w_v)
            pltpu.async_copy(feat_hbm.at[src_v], rows_v, sem).wait()

            def group_body(g, gcarry):
                wv16 = w_v[pl.ds(g * LANES, LANES)]
                for e16 in range(LANES):
                    wbc = lax.gather(
                        wv16,
                        jnp.full((LANES, 1), e16, jnp.int32),
                        dimension_numbers=lax.GatherDimensionNumbers(
                            offset_dims=(), collapsed_slice_dims=(0,),
                            start_index_map=(0,)),
                        slice_sizes=(1,),
                        mode=lax.GatherScatterMode.PROMISE_IN_BOUNDS)
                    e = g * LANES + e16
                    for j in range(d_vecs):
                        sl = pl.ds(j * LANES, LANES)
                        rows_v[e, sl] = rows_v[e, sl] * wbc
                return gcarry
            lax.fori_loop(0, CHUNK // LANES, group_body, 0)

            pltpu.sync_copy(rows_v, acc.at[dst_v], add=True)
            return carry
        lax.fori_loop(0, n_chunks, chunk_body, 0)
        plsc.subcore_barrier()

        pltpu.sync_copy(acc.at[pl.ds(r0, rows_w), :],
                        out_hbm.at[c, pl.ds(r0, rows_w), :])

    return agg(features, src, dst, wgt)


def _tc_combine_matmul_relu(partials, W):
    _, n_pad, d = partials.shape
    d_out = W.shape[1]
    blk = 1024
    assert n_pad % blk == 0

    def body(p_ref, w_ref, o_ref):
        pp = p_ref[0] + p_ref[1]
        acc = jnp.dot(pp, w_ref[...], preferred_element_type=jnp.float32,
                      precision=lax.Precision.HIGHEST)
        o_ref[...] = jnp.maximum(acc, 0.0)

    return pl.pallas_call(
        body,
        grid=(n_pad // blk,),
        in_specs=[
            pl.BlockSpec((NC, blk, d), lambda i: (0, i, 0)),
            pl.BlockSpec((d, d_out), lambda i: (0, 0)),
        ],
        out_specs=pl.BlockSpec((blk, d_out), lambda i: (i, 0)),
        out_shape=jax.ShapeDtypeStruct((n_pad, d_out), jnp.float32),
    )(partials, W)


def kernel(features, edge_index, edge_weight, W):
    n_nodes = features.shape[0]
    n_pad = 10240
    dst = edge_index[0].astype(jnp.int32)
    src = edge_index[1].astype(jnp.int32)
    wgt = edge_weight.astype(jnp.float32)
    partials = _sc_aggregate(features, src, dst, wgt, n_pad)
    return _tc_combine_matmul_relu(partials, W)[:n_nodes]


# speedup vs baseline: 8.2102x; 1.8267x over previous
import functools

import jax
import jax.numpy as jnp
from jax import lax
from jax.experimental import pallas as pl
from jax.experimental.pallas import tpu as pltpu
from jax.experimental.pallas import tpu_sc as plsc

NC = 2
NS = 16
NW = NC * NS
LANES = 16
CHUNK = 80


def _sc_aggregate(features, src, dst, wgt, n_pad):
    n_nodes, d = features.shape
    n_edges = src.shape[0]
    assert n_edges % NW == 0
    e_w = n_edges // NW
    assert e_w % CHUNK == 0
    n_chunks = e_w // CHUNK
    assert n_chunks >= 6 and n_chunks % 2 == 1
    assert n_pad % (NS * 8) == 0
    rows_w = n_pad // NS
    zrows = 128
    assert rows_w % zrows == 0
    d_vecs = d // LANES
    groups = CHUNK // LANES

    mesh = plsc.VectorSubcoreMesh(core_axis_name="c", subcore_axis_name="s")

    @functools.partial(
        pl.kernel,
        out_type=jax.ShapeDtypeStruct((NC, n_pad, d), jnp.float32),
        mesh=mesh,
        scratch_types=[
            pltpu.VMEM((2, CHUNK), jnp.int32),
            pltpu.VMEM((2, CHUNK), jnp.int32),
            pltpu.VMEM((2, CHUNK), jnp.float32),
            pltpu.VMEM((2, CHUNK), jnp.int32),
            pltpu.VMEM((2, CHUNK), jnp.float32),
            pltpu.VMEM((2, CHUNK, d), jnp.float32),
            pltpu.VMEM((zrows, d), jnp.float32),
            pltpu.VMEM_SHARED((n_pad, d), jnp.float32),
            pltpu.SemaphoreType.DMA,
            pltpu.SemaphoreType.DMA,
            pltpu.SemaphoreType.DMA,
            pltpu.SemaphoreType.DMA,
            pltpu.SemaphoreType.DMA,
            pltpu.SemaphoreType.DMA,
        ],
    )
    def agg(feat_hbm, src_hbm, dst_hbm, wgt_hbm, out_hbm,
            src_f, dst_f, w_f, sdst, sw, rows, zbuf, acc,
            sem_i0, sem_i1, sem_g0, sem_g1, sem_s0, sem_s1):
        c = lax.axis_index("c")
        s = lax.axis_index("s")
        wid = c * NS + s
        e0 = wid * e_w
        sem_i = (sem_i0, sem_i1)
        sem_g = (sem_g0, sem_g1)
        sem_s = (sem_s0, sem_s1)

        def idx_start(i, p):
            base = e0 + i * CHUNK
            pltpu.async_copy(src_hbm.at[pl.ds(base, CHUNK)], src_f.at[p], sem_i[p])
            pltpu.async_copy(dst_hbm.at[pl.ds(base, CHUNK)], dst_f.at[p], sem_i[p])
            pltpu.async_copy(wgt_hbm.at[pl.ds(base, CHUNK)], w_f.at[p], sem_i[p])

        def idx_wait(p):
            pltpu.make_async_copy(src_hbm.at[pl.ds(0, CHUNK)], src_f.at[p], sem_i[p]).wait()
            pltpu.make_async_copy(dst_hbm.at[pl.ds(0, CHUNK)], dst_f.at[p], sem_i[p]).wait()
            pltpu.make_async_copy(wgt_hbm.at[pl.ds(0, CHUNK)], w_f.at[p], sem_i[p]).wait()

        def gather_start(p):
            pltpu.async_copy(feat_hbm.at[src_f.at[p]], rows.at[p], sem_g[p])

        def gather_wait(p):
            pltpu.make_async_copy(feat_hbm.at[src_f.at[p]], rows.at[p], sem_g[p]).wait()

        def scat_start(p):
            pltpu.async_copy(rows.at[p], acc.at[sdst.at[p]], sem_s[p], add=True)

        def scat_wait(p):
            pltpu.make_async_copy(rows.at[p], acc.at[sdst.at[p]], sem_s[p]).wait()

        def stash_idx(p):
            for g in range(groups):
                sl = pl.ds(g * LANES, LANES)
                sdst[p, sl] = dst_f[p, sl]
                sw[p, sl] = w_f[p, sl]

        def compute(p):
            def group_body(g, carry):
                wv16 = sw[p, pl.ds(g * LANES, LANES)]
                for e16 in range(LANES):
                    wbc = lax.gather(
                        wv16,
                        jnp.full((LANES, 1), e16, jnp.int32),
                        dimension_numbers=lax.GatherDimensionNumbers(
                            offset_dims=(), collapsed_slice_dims=(0,),
                            start_index_map=(0,)),
                        slice_sizes=(1,),
                        mode=lax.GatherScatterMode.PROMISE_IN_BOUNDS)
                    e = g * LANES + e16
                    for j in range(d_vecs):
                        sl = pl.ds(j * LANES, LANES)
                        rows[p, e, sl] = rows[p, e, sl] * wbc
                return carry
            lax.fori_loop(0, groups, group_body, 0)

        def emit_iter(i, p, first=False, has_next=True, has_next2=True):
            gather_wait(p)
            stash_idx(p)
            if has_next2:
                idx_start(i + 2, p)
            compute(p)
            scat_start(p)
            if has_next:
                if not first:
                    scat_wait(1 - p)
                idx_wait(1 - p)
                gather_start(1 - p)

        idx_start(0, 0)
        idx_start(1, 1)

        def zfill(i, carry):
            for j in range(d_vecs):
                zbuf[i, pl.ds(j * LANES, LANES)] = jnp.zeros((LANES,), jnp.float32)
            return carry
        lax.fori_loop(0, zrows, zfill, 0)
        r0 = s * rows_w

        idx_wait(0)
        gather_start(0)

        def zcopy(i, carry):
            pltpu.sync_copy(zbuf, acc.at[pl.ds(r0 + i * zrows, zrows), :])
            return carry
        lax.fori_loop(0, rows_w // zrows, zcopy, 0)
        plsc.subcore_barrier()

        emit_iter(0, 0, first=True)
        emit_iter(1, 1)

        def steady(k, carry):
            i = 2 * k
            emit_iter(i, 0)
            emit_iter(i + 1, 1)
            return carry
        lax.fori_loop(1, (n_chunks - 3) // 2, steady, 0)

        emit_iter(n_chunks - 3, 0)
        emit_iter(n_chunks - 2, 1, has_next2=False)
        emit_iter(n_chunks - 1, 0, has_next=False, has_next2=False)
        scat_wait(1)
        scat_wait(0)
        plsc.subcore_barrier()

        pltpu.sync_copy(acc.at[pl.ds(r0, rows_w), :],
                        out_hbm.at[c, pl.ds(r0, rows_w), :])

    return agg(features, src, dst, wgt)


def _tc_combine_matmul_relu(partials, W):
    _, n_pad, d = partials.shape
    d_out = W.shape[1]
    blk = 1024
    assert n_pad % blk == 0

    def body(p_ref, w_ref, o_ref):
        pp = p_ref[0] + p_ref[1]
        acc = jnp.dot(pp, w_ref[...], preferred_element_type=jnp.float32,
                      precision=lax.Precision.HIGHEST)
        o_ref[...] = jnp.maximum(acc, 0.0)

    return pl.pallas_call(
        body,
        grid=(n_pad // blk,),
        in_specs=[
            pl.BlockSpec((NC, blk, d), lambda i: (0, i, 0)),
            pl.BlockSpec((d, d_out), lambda i: (0, 0)),
        ],
        out_specs=pl.BlockSpec((blk, d_out), lambda i: (i, 0)),
        out_shape=jax.ShapeDtypeStruct((n_pad, d_out), jnp.float32),
    )(partials, W)


def kernel(features, edge_index, edge_weight, W):
    n_nodes = features.shape[0]
    n_pad = 10240
    dst = edge_index[0].astype(jnp.int32)
    src = edge_index[1].astype(jnp.int32)
    wgt = edge_weight.astype(jnp.float32)
    partials = _sc_aggregate(features, src, dst, wgt, n_pad)
    return _tc_combine_matmul_relu(partials, W)[:n_nodes]


# speedup vs baseline: 10.1899x; 1.2411x over previous
import functools

import jax
import jax.numpy as jnp
from jax import lax
from jax.experimental import pallas as pl
from jax.experimental.pallas import tpu as pltpu
from jax.experimental.pallas import tpu_sc as plsc

NC = 2
NS = 16
NW = NC * NS
LANES = 16
CHUNK = 80
RING = 3


def _sc_aggregate(features, src, dst, wgt, n_pad):
    n_nodes, d = features.shape
    n_edges = src.shape[0]
    assert n_edges % NW == 0
    e_w = n_edges // NW
    assert e_w % CHUNK == 0
    n_chunks = e_w // CHUNK
    assert n_chunks >= RING
    n_iters = -(-(n_chunks + 1) // RING) * RING
    assert n_iters == n_chunks + 1
    assert n_pad % (NS * 8) == 0
    rows_w = n_pad // NS
    zrows = 40
    assert rows_w % zrows == 0
    d_vecs = d // LANES
    groups = CHUNK // LANES

    mesh = plsc.VectorSubcoreMesh(core_axis_name="c", subcore_axis_name="s")

    @functools.partial(
        pl.kernel,
        out_type=jax.ShapeDtypeStruct((NC, n_pad, d), jnp.float32),
        mesh=mesh,
        scratch_types=[
            pltpu.VMEM((RING, CHUNK), jnp.int32),
            pltpu.VMEM((RING, CHUNK), jnp.int32),
            pltpu.VMEM((RING, CHUNK), jnp.float32),
            pltpu.VMEM((RING, CHUNK), jnp.int32),
            pltpu.VMEM((RING, CHUNK), jnp.float32),
            pltpu.VMEM((RING, CHUNK, d), jnp.float32),
            pltpu.VMEM((zrows, d), jnp.float32),
            pltpu.VMEM_SHARED((n_pad, d), jnp.float32),
            [pltpu.SemaphoreType.DMA] * RING,
            [pltpu.SemaphoreType.DMA] * RING,
            [pltpu.SemaphoreType.DMA] * RING,
        ],
    )
    def agg(feat_hbm, src_hbm, dst_hbm, wgt_hbm, out_hbm,
            src_f, dst_f, w_f, sdst, sw, rows, zbuf, acc,
            sem_i, sem_g, sem_s):
        c = lax.axis_index("c")
        s = lax.axis_index("s")
        wid = c * NS + s
        e0 = wid * e_w

        def idx_start(i, r):
            base = e0 + i * CHUNK
            pltpu.async_copy(src_hbm.at[pl.ds(base, CHUNK)], src_f.at[r], sem_i[r])
            pltpu.async_copy(dst_hbm.at[pl.ds(base, CHUNK)], dst_f.at[r], sem_i[r])
            pltpu.async_copy(wgt_hbm.at[pl.ds(base, CHUNK)], w_f.at[r], sem_i[r])

        def idx_wait(r):
            pltpu.make_async_copy(src_hbm.at[pl.ds(0, CHUNK)], src_f.at[r], sem_i[r]).wait()
            pltpu.make_async_copy(dst_hbm.at[pl.ds(0, CHUNK)], dst_f.at[r], sem_i[r]).wait()
            pltpu.make_async_copy(wgt_hbm.at[pl.ds(0, CHUNK)], w_f.at[r], sem_i[r]).wait()

        def gather_start(r):
            pltpu.async_copy(feat_hbm.at[src_f.at[r]], rows.at[r], sem_g[r])

        def gather_wait(r):
            pltpu.make_async_copy(feat_hbm.at[src_f.at[r]], rows.at[r], sem_g[r]).wait()

        def scat_start(r):
            pltpu.async_copy(rows.at[r], acc.at[sdst.at[r]], sem_s[r], add=True)

        def scat_wait(r):
            pltpu.make_async_copy(rows.at[r], acc.at[sdst.at[r]], sem_s[r]).wait()

        def stash_idx(r):
            for g in range(groups):
                sl = pl.ds(g * LANES, LANES)
                sdst[r, sl] = dst_f[r, sl]
                sw[r, sl] = w_f[r, sl]

        def compute(r):
            def group_body(g, carry):
                wv16 = sw[r, pl.ds(g * LANES, LANES)]
                for e16 in range(LANES):
                    wbc = lax.gather(
                        wv16,
                        jnp.full((LANES, 1), e16, jnp.int32),
                        dimension_numbers=lax.GatherDimensionNumbers(
                            offset_dims=(), collapsed_slice_dims=(0,),
                            start_index_map=(0,)),
                        slice_sizes=(1,),
                        mode=lax.GatherScatterMode.PROMISE_IN_BOUNDS)
                    e = g * LANES + e16
                    for j in range(d_vecs):
                        sl = pl.ds(j * LANES, LANES)
                        rows[r, e, sl] = rows[r, e, sl] * wbc
                return carry
            lax.fori_loop(0, groups, group_body, 0)

        idx_start(0, 0)
        idx_start(1, 1)

        def zfill(i, carry):
            for j in range(d_vecs):
                zbuf[i, pl.ds(j * LANES, LANES)] = jnp.zeros((LANES,), jnp.float32)
            return carry
        lax.fori_loop(0, zrows, zfill, 0)
        r0 = s * rows_w

        idx_wait(0)
        gather_start(0)

        def zcopy(i, carry):
            pltpu.sync_copy(zbuf, acc.at[pl.ds(r0 + i * zrows, zrows), :])
            return carry
        lax.fori_loop(0, rows_w // zrows, zcopy, 0)
        plsc.subcore_barrier()

        def step(k, carry):
            for j in range(RING):
                i = k * RING + j
                live = i < n_chunks

                @pl.when(live)
                def _():
                    gather_wait(j)
                    stash_idx(j)

                @pl.when(i + 2 < n_chunks)
                def _():
                    idx_start(i + 2, (j + 2) % RING)

                @pl.when(jnp.logical_and(i >= 2, i - 2 < n_chunks))
                def _():
                    scat_wait((j + 1) % RING)

                @pl.when(i + 1 < n_chunks)
                def _():
                    idx_wait((j + 1) % RING)
                    gather_start((j + 1) % RING)

                @pl.when(live)
                def _():
                    compute(j)
                    scat_start(j)
            return carry
        lax.fori_loop(0, n_iters // RING, step, 0)
        scat_wait((n_chunks - 1) % RING)
        plsc.subcore_barrier()

        pltpu.sync_copy(acc.at[pl.ds(r0, rows_w), :],
                        out_hbm.at[c, pl.ds(r0, rows_w), :])

    return agg(features, src, dst, wgt)


def _tc_combine_matmul_relu(partials, W, n_nodes):
    _, n_pad, d = partials.shape
    d_out = W.shape[1]
    blk = 1000
    assert n_nodes % blk == 0

    def body(p_ref, w_ref, o_ref):
        pp = p_ref[0] + p_ref[1]
        acc = jnp.dot(pp, w_ref[...], preferred_element_type=jnp.float32,
                      precision=lax.Precision.HIGHEST)
        o_ref[...] = jnp.maximum(acc, 0.0)

    return pl.pallas_call(
        body,
        grid=(n_nodes // blk,),
        in_specs=[
            pl.BlockSpec((NC, blk, d), lambda i: (0, i, 0)),
            pl.BlockSpec((d, d_out), lambda i: (0, 0)),
        ],
        out_specs=pl.BlockSpec((blk, d_out), lambda i: (i, 0)),
        out_shape=jax.ShapeDtypeStruct((n_nodes, d_out), jnp.float32),
    )(partials, W)


def kernel(features, edge_index, edge_weight, W):
    n_nodes = features.shape[0]
    n_pad = 10240
    dst = edge_index[0].astype(jnp.int32)
    src = edge_index[1].astype(jnp.int32)
    wgt = edge_weight.astype(jnp.float32)
    partials = _sc_aggregate(features, src, dst, wgt, n_pad)
    return _tc_combine_matmul_relu(partials, W, n_nodes)


# speedup vs baseline: 10.7886x; 1.0588x over previous
import functools

import jax
import jax.numpy as jnp
from jax import lax
from jax.experimental import pallas as pl
from jax.experimental.pallas import tpu as pltpu
from jax.experimental.pallas import tpu_sc as plsc

NC = 2
NS = 16
NW = NC * NS
LANES = 16
CHUNK = 80
RING = 3


def _sc_aggregate(features, eflat, wgt, n_pad):
    n_nodes, d = features.shape
    n_edges = eflat.shape[0] // 2
    assert n_edges % NW == 0
    e_w = n_edges // NW
    assert e_w % CHUNK == 0
    n_chunks = e_w // CHUNK
    assert n_chunks >= RING
    n_iters = -(-(n_chunks + 1) // RING) * RING
    assert n_iters == n_chunks + 1
    assert n_pad % (NS * 8) == 0
    rows_w = n_pad // NS
    zrows = 40
    assert rows_w % zrows == 0
    d_vecs = d // LANES
    groups = CHUNK // LANES

    mesh = plsc.VectorSubcoreMesh(core_axis_name="c", subcore_axis_name="s")

    @functools.partial(
        pl.kernel,
        out_type=jax.ShapeDtypeStruct((NC, n_pad, d), jnp.float32),
        mesh=mesh,
        scratch_types=[
            pltpu.VMEM((RING, CHUNK), jnp.int32),
            pltpu.VMEM((RING, CHUNK), jnp.int32),
            pltpu.VMEM((RING, CHUNK), jnp.float32),
            pltpu.VMEM((RING, CHUNK), jnp.int32),
            pltpu.VMEM((RING, CHUNK), jnp.float32),
            pltpu.VMEM((RING, CHUNK, d), jnp.float32),
            pltpu.VMEM((zrows, d), jnp.float32),
            pltpu.VMEM_SHARED((n_pad, d), jnp.float32),
            [pltpu.SemaphoreType.DMA] * RING,
            [pltpu.SemaphoreType.DMA] * RING,
            [pltpu.SemaphoreType.DMA] * RING,
        ],
    )
    def agg(feat_hbm, eflat_hbm, wgt_hbm, out_hbm,
            src_f, dst_f, w_f, sdst, sw, rows, zbuf, acc,
            sem_i, sem_g, sem_s):
        c = lax.axis_index("c")
        s = lax.axis_index("s")
        wid = c * NS + s
        e0 = wid * e_w

        def idx_start(i, r):
            base = e0 + i * CHUNK
            pltpu.async_copy(eflat_hbm.at[pl.ds(n_edges + base, CHUNK)], src_f.at[r], sem_i[r])
            pltpu.async_copy(eflat_hbm.at[pl.ds(base, CHUNK)], dst_f.at[r], sem_i[r])
            pltpu.async_copy(wgt_hbm.at[pl.ds(base, CHUNK)], w_f.at[r], sem_i[r])

        def idx_wait(r):
            pltpu.make_async_copy(eflat_hbm.at[pl.ds(0, CHUNK)], src_f.at[r], sem_i[r]).wait()
            pltpu.make_async_copy(eflat_hbm.at[pl.ds(0, CHUNK)], dst_f.at[r], sem_i[r]).wait()
            pltpu.make_async_copy(wgt_hbm.at[pl.ds(0, CHUNK)], w_f.at[r], sem_i[r]).wait()

        def gather_start(r):
            pltpu.async_copy(feat_hbm.at[src_f.at[r]], rows.at[r], sem_g[r])

        def gather_wait(r):
            pltpu.make_async_copy(feat_hbm.at[src_f.at[r]], rows.at[r], sem_g[r]).wait()

        def scat_start(r):
            pltpu.async_copy(rows.at[r], acc.at[sdst.at[r]], sem_s[r], add=True)

        def scat_wait(r):
            pltpu.make_async_copy(rows.at[r], acc.at[sdst.at[r]], sem_s[r]).wait()

        def stash_idx(r):
            for g in range(groups):
                sl = pl.ds(g * LANES, LANES)
                sdst[r, sl] = dst_f[r, sl]
                sw[r, sl] = w_f[r, sl]

        def compute(r):
            def group_body(g, carry):
                wv16 = sw[r, pl.ds(g * LANES, LANES)]
                for e16 in range(LANES):
                    wbc = lax.gather(
                        wv16,
                        jnp.full((LANES, 1), e16, jnp.int32),
                        dimension_numbers=lax.GatherDimensionNumbers(
                            offset_dims=(), collapsed_slice_dims=(0,),
                            start_index_map=(0,)),
                        slice_sizes=(1,),
                        mode=lax.GatherScatterMode.PROMISE_IN_BOUNDS)
                    e = g * LANES + e16
                    for j in range(d_vecs):
                        sl = pl.ds(j * LANES, LANES)
                        rows[r, e, sl] = rows[r, e, sl] * wbc
                return carry
            lax.fori_loop(0, groups, group_body, 0)

        idx_start(0, 0)
        idx_start(1, 1)

        def zfill(i, carry):
            for j in range(d_vecs):
                zbuf[i, pl.ds(j * LANES, LANES)] = jnp.zeros((LANES,), jnp.float32)
            return carry
        lax.fori_loop(0, zrows, zfill, 0)
        r0 = s * rows_w

        idx_wait(0)
        gather_start(0)

        def zcopy(i, carry):
            pltpu.sync_copy(zbuf, acc.at[pl.ds(r0 + i * zrows, zrows), :])
            return carry
        lax.fori_loop(0, rows_w // zrows, zcopy, 0)
        plsc.subcore_barrier()

        def step(k, carry):
            for j in range(RING):
                i = k * RING + j
                live = i < n_chunks

                @pl.when(live)
                def _():
                    gather_wait(j)
                    stash_idx(j)

                @pl.when(i + 2 < n_chunks)
                def _():
                    idx_start(i + 2, (j + 2) % RING)

                @pl.when(jnp.logical_and(i >= 2, i - 2 < n_chunks))
                def _():
                    scat_wait((j + 1) % RING)

                @pl.when(i + 1 < n_chunks)
                def _():
                    idx_wait((j + 1) % RING)
                    gather_start((j + 1) % RING)

                @pl.when(live)
                def _():
                    compute(j)
                    scat_start(j)
            return carry
        lax.fori_loop(0, n_iters // RING, step, 0)
        scat_wait((n_chunks - 1) % RING)
        plsc.subcore_barrier()

        pltpu.sync_copy(acc.at[pl.ds(r0, rows_w), :],
                        out_hbm.at[c, pl.ds(r0, rows_w), :])

    return agg(features, eflat, wgt)


def _tc_combine_matmul_relu(partials, W, n_nodes):
    _, n_pad, d = partials.shape
    d_out = W.shape[1]
    blk = 1000
    assert n_nodes % blk == 0

    def body(p_ref, w_ref, o_ref):
        pp = p_ref[0] + p_ref[1]
        acc = jnp.dot(pp, w_ref[...], preferred_element_type=jnp.float32,
                      precision=lax.Precision.HIGHEST)
        o_ref[...] = jnp.maximum(acc, 0.0)

    return pl.pallas_call(
        body,
        grid=(n_nodes // blk,),
        in_specs=[
            pl.BlockSpec((NC, blk, d), lambda i: (0, i, 0)),
            pl.BlockSpec((d, d_out), lambda i: (0, 0)),
        ],
        out_specs=pl.BlockSpec((blk, d_out), lambda i: (i, 0)),
        out_shape=jax.ShapeDtypeStruct((n_nodes, d_out), jnp.float32),
    )(partials, W)


def kernel(features, edge_index, edge_weight, W):
    n_nodes = features.shape[0]
    n_pad = 10240
    eflat = edge_index.astype(jnp.int32).reshape(-1)
    wgt = edge_weight.astype(jnp.float32)
    partials = _sc_aggregate(features, eflat, wgt, n_pad)
    return _tc_combine_matmul_relu(partials, W, n_nodes)


# speedup vs baseline: 13.8953x; 1.2880x over previous
import functools

import jax
import jax.numpy as jnp
from jax import lax
from jax.experimental import pallas as pl
from jax.experimental.pallas import tpu as pltpu
from jax.experimental.pallas import tpu_sc as plsc

NC = 2
NS = 16
NW = NC * NS
LANES = 16
CHUNK = 80
RING = 4


def _sc_aggregate(features, eflat, wgt, n_pad):
    n_nodes, d = features.shape
    n_edges = eflat.shape[0] // 2
    assert n_edges % NW == 0
    e_w = n_edges // NW
    assert e_w % CHUNK == 0
    n_chunks = e_w // CHUNK
    assert n_chunks >= RING
    n_iters = -(-(n_chunks + 2) // RING) * RING
    assert n_iters >= n_chunks + 2
    assert n_pad % (NS * 8) == 0
    rows_w = n_pad // NS
    assert rows_w % CHUNK == 0
    d_vecs = d // LANES
    groups = CHUNK // LANES

    mesh = plsc.VectorSubcoreMesh(core_axis_name="c", subcore_axis_name="s")

    @functools.partial(
        pl.kernel,
        out_type=jax.ShapeDtypeStruct((NC, n_pad, d), jnp.float32),
        mesh=mesh,
        scratch_types=[
            pltpu.VMEM((RING, CHUNK), jnp.int32),
            pltpu.VMEM((RING, CHUNK), jnp.int32),
            pltpu.VMEM((RING, CHUNK), jnp.float32),
            pltpu.VMEM((RING, CHUNK), jnp.int32),
            pltpu.VMEM((RING, CHUNK), jnp.float32),
            pltpu.VMEM((RING, CHUNK, d), jnp.float32),
            pltpu.VMEM_SHARED((n_pad, d), jnp.float32),
            [pltpu.SemaphoreType.DMA] * RING,
            [pltpu.SemaphoreType.DMA] * RING,
            [pltpu.SemaphoreType.DMA] * RING,
        ],
    )
    def agg(feat_hbm, eflat_hbm, wgt_hbm, out_hbm,
            src_f, dst_f, w_f, sdst, sw, rows, acc,
            sem_i, sem_g, sem_s):
        c = lax.axis_index("c")
        s = lax.axis_index("s")
        wid = c * NS + s
        e0 = wid * e_w

        def idx_start(i, r):
            base = e0 + i * CHUNK
            pltpu.async_copy(eflat_hbm.at[pl.ds(n_edges + base, CHUNK)], src_f.at[r], sem_i[r])
            pltpu.async_copy(eflat_hbm.at[pl.ds(base, CHUNK)], dst_f.at[r], sem_i[r])
            pltpu.async_copy(wgt_hbm.at[pl.ds(base, CHUNK)], w_f.at[r], sem_i[r])

        def idx_wait(r):
            pltpu.make_async_copy(eflat_hbm.at[pl.ds(0, CHUNK)], src_f.at[r], sem_i[r]).wait()
            pltpu.make_async_copy(eflat_hbm.at[pl.ds(0, CHUNK)], dst_f.at[r], sem_i[r]).wait()
            pltpu.make_async_copy(wgt_hbm.at[pl.ds(0, CHUNK)], w_f.at[r], sem_i[r]).wait()

        def gather_start(r):
            pltpu.async_copy(feat_hbm.at[src_f.at[r]], rows.at[r], sem_g[r])

        def gather_wait(r):
            pltpu.make_async_copy(feat_hbm.at[src_f.at[r]], rows.at[r], sem_g[r]).wait()

        def scat_start(r):
            pltpu.async_copy(rows.at[r], acc.at[sdst.at[r]], sem_s[r], add=True)

        def scat_wait(r):
            pltpu.make_async_copy(rows.at[r], acc.at[sdst.at[r]], sem_s[r]).wait()

        def stash_idx(r):
            for g in range(groups):
                sl = pl.ds(g * LANES, LANES)
                sdst[r, sl] = dst_f[r, sl]
                sw[r, sl] = w_f[r, sl]

        def compute(r):
            def group_body(g, carry):
                wv16 = sw[r, pl.ds(g * LANES, LANES)]
                for e16 in range(LANES):
                    wbc = lax.gather(
                        wv16,
                        jnp.full((LANES, 1), e16, jnp.int32),
                        dimension_numbers=lax.GatherDimensionNumbers(
                            offset_dims=(), collapsed_slice_dims=(0,),
                            start_index_map=(0,)),
                        slice_sizes=(1,),
                        mode=lax.GatherScatterMode.PROMISE_IN_BOUNDS)
                    e = g * LANES + e16
                    for j in range(d_vecs):
                        sl = pl.ds(j * LANES, LANES)
                        rows[r, e, sl] = rows[r, e, sl] * wbc
                return carry
            lax.fori_loop(0, groups, group_body, 0)

        idx_start(0, 0)
        idx_start(1, 1)
        idx_start(2, 2)

        def zfill(i, carry):
            for j in range(d_vecs):
                rows[0, i, pl.ds(j * LANES, LANES)] = jnp.zeros((LANES,), jnp.float32)
            return carry
        lax.fori_loop(0, CHUNK, zfill, 0)
        r0 = s * rows_w

        def zcopy(i, carry):
            pltpu.sync_copy(rows.at[0], acc.at[pl.ds(r0 + i * CHUNK, CHUNK), :])
            return carry
        lax.fori_loop(0, rows_w // CHUNK, zcopy, 0)

        idx_wait(0)
        gather_start(0)
        idx_wait(1)
        gather_start(1)
        plsc.subcore_barrier()

        def step(k, carry):
            for j in range(RING):
                i = k * RING + j
                live = i < n_chunks

                @pl.when(live)
                def _():
                    gather_wait(j)
                    stash_idx(j)

                @pl.when(i + 3 < n_chunks)
                def _():
                    idx_start(i + 3, (j + 3) % RING)

                @pl.when(jnp.logical_and(i >= 2, i - 2 < n_chunks))
                def _():
                    scat_wait((j + 2) % RING)

                @pl.when(i + 2 < n_chunks)
                def _():
                    idx_wait((j + 2) % RING)
                    gather_start((j + 2) % RING)

                @pl.when(live)
                def _():
                    compute(j)
                    scat_start(j)
            return carry
        lax.fori_loop(0, n_iters // RING, step, 0)
        plsc.subcore_barrier()

        pltpu.sync_copy(acc.at[pl.ds(r0, rows_w), :],
                        out_hbm.at[c, pl.ds(r0, rows_w), :])

    return agg(features, eflat, wgt)


def _tc_combine_matmul_relu(partials, W, n_nodes):
    _, n_pad, d = partials.shape
    d_out = W.shape[1]
    blk = 1000
    assert n_nodes % blk == 0

    def body(p_ref, w_ref, o_ref):
        pp = p_ref[0] + p_ref[1]
        acc = jnp.dot(pp, w_ref[...], preferred_element_type=jnp.float32,
                      precision=lax.Precision.HIGHEST)
        o_ref[...] = jnp.maximum(acc, 0.0)

    return pl.pallas_call(
        body,
        grid=(n_nodes // blk,),
        in_specs=[
            pl.BlockSpec((NC, blk, d), lambda i: (0, i, 0)),
            pl.BlockSpec((d, d_out), lambda i: (0, 0)),
        ],
        out_specs=pl.BlockSpec((blk, d_out), lambda i: (i, 0)),
        out_shape=jax.ShapeDtypeStruct((n_nodes, d_out), jnp.float32),
    )(partials, W)


def kernel(features, edge_index, edge_weight, W):
    n_nodes = features.shape[0]
    n_pad = 10240
    eflat = edge_index.astype(jnp.int32).reshape(-1)
    wgt = edge_weight.astype(jnp.float32)
    partials = _sc_aggregate(features, eflat, wgt, n_pad)
    return _tc_combine_matmul_relu(partials, W, n_nodes)


# speedup vs baseline: 13.9112x; 1.0011x over previous
import functools

import jax
import jax.numpy as jnp
from jax import lax
from jax.experimental import pallas as pl
from jax.experimental.pallas import tpu as pltpu
from jax.experimental.pallas import tpu_sc as plsc

NC = 2
NS = 16
NW = NC * NS
LANES = 16
CHUNK = 80
RING = 4


def _sc_aggregate(features, eflat, wgt, n_pad):
    n_nodes, d = features.shape
    n_edges = eflat.shape[0] // 2
    assert n_edges % NW == 0
    e_w = n_edges // NW
    assert e_w % CHUNK == 0
    n_chunks = e_w // CHUNK
    assert n_chunks >= RING
    n_iters = -(-(n_chunks + 2) // RING) * RING
    assert n_iters >= n_chunks + 2
    assert n_pad % (NS * 8) == 0
    rows_w = n_pad // NS
    assert rows_w % CHUNK == 0
    d_vecs = d // LANES
    groups = CHUNK // LANES

    mesh = plsc.VectorSubcoreMesh(core_axis_name="c", subcore_axis_name="s")

    @functools.partial(
        pl.kernel,
        out_type=jax.ShapeDtypeStruct((NC, n_pad, d), jnp.float32),
        mesh=mesh,
        scratch_types=[
            pltpu.VMEM((RING, CHUNK), jnp.int32),
            pltpu.VMEM((RING, CHUNK), jnp.int32),
            pltpu.VMEM((RING, CHUNK), jnp.float32),
            pltpu.VMEM((RING, CHUNK), jnp.int32),
            pltpu.VMEM((RING, CHUNK), jnp.float32),
            pltpu.VMEM((RING, CHUNK, d), jnp.float32),
            pltpu.VMEM_SHARED((n_pad, d), jnp.float32),
            [pltpu.SemaphoreType.DMA] * RING,
            [pltpu.SemaphoreType.DMA] * RING,
            [pltpu.SemaphoreType.DMA] * RING,
        ],
    )
    def agg(feat_hbm, eflat_hbm, wgt_hbm, out_hbm,
            src_f, dst_f, w_f, sdst, sw, rows, acc,
            sem_i, sem_g, sem_s):
        c = lax.axis_index("c")
        s = lax.axis_index("s")
        wid = c * NS + s
        e0 = wid * e_w

        def idx_start(i, r):
            base = e0 + i * CHUNK
            pltpu.async_copy(eflat_hbm.at[pl.ds(n_edges + base, CHUNK)], src_f.at[r], sem_i[r])
            pltpu.async_copy(eflat_hbm.at[pl.ds(base, CHUNK)], dst_f.at[r], sem_i[r])
            pltpu.async_copy(wgt_hbm.at[pl.ds(base, CHUNK)], w_f.at[r], sem_i[r])

        def idx_wait(r):
            pltpu.make_async_copy(eflat_hbm.at[pl.ds(0, CHUNK)], src_f.at[r], sem_i[r]).wait()
            pltpu.make_async_copy(eflat_hbm.at[pl.ds(0, CHUNK)], dst_f.at[r], sem_i[r]).wait()
            pltpu.make_async_copy(wgt_hbm.at[pl.ds(0, CHUNK)], w_f.at[r], sem_i[r]).wait()

        half = CHUNK // 2

        def gather_start(r):
            pltpu.async_copy(feat_hbm.at[src_f.at[r, pl.ds(0, half)]],
                             rows.at[r, pl.ds(0, half), :], sem_g[r])
            pltpu.async_copy(feat_hbm.at[src_f.at[r, pl.ds(half, half)]],
                             rows.at[r, pl.ds(half, half), :], sem_g[r])

        def gather_wait(r):
            pltpu.make_async_copy(feat_hbm.at[src_f.at[r, pl.ds(0, half)]],
                                  rows.at[r, pl.ds(0, half), :], sem_g[r]).wait()
            pltpu.make_async_copy(feat_hbm.at[src_f.at[r, pl.ds(half, half)]],
                                  rows.at[r, pl.ds(half, half), :], sem_g[r]).wait()

        def scat_start(r):
            pltpu.async_copy(rows.at[r], acc.at[sdst.at[r]], sem_s[r], add=True)

        def scat_wait(r):
            pltpu.make_async_copy(rows.at[r], acc.at[sdst.at[r]], sem_s[r]).wait()

        def stash_idx(r):
            for g in range(groups):
                sl = pl.ds(g * LANES, LANES)
                sdst[r, sl] = dst_f[r, sl]
                sw[r, sl] = w_f[r, sl]

        def compute(r):
            def group_body(g, carry):
                wv16 = sw[r, pl.ds(g * LANES, LANES)]
                for e16 in range(LANES):
                    wbc = lax.gather(
                        wv16,
                        jnp.full((LANES, 1), e16, jnp.int32),
                        dimension_numbers=lax.GatherDimensionNumbers(
                            offset_dims=(), collapsed_slice_dims=(0,),
                            start_index_map=(0,)),
                        slice_sizes=(1,),
                        mode=lax.GatherScatterMode.PROMISE_IN_BOUNDS)
                    e = g * LANES + e16
                    for j in range(d_vecs):
                        sl = pl.ds(j * LANES, LANES)
                        rows[r, e, sl] = rows[r, e, sl] * wbc
                return carry
            lax.fori_loop(0, groups, group_body, 0)

        idx_start(0, 0)
        idx_start(1, 1)
        idx_start(2, 2)

        def zfill(i, carry):
            for j in range(d_vecs):
                rows[0, i, pl.ds(j * LANES, LANES)] = jnp.zeros((LANES,), jnp.float32)
            return carry
        lax.fori_loop(0, CHUNK, zfill, 0)
        r0 = s * rows_w

        def zcopy(i, carry):
            pltpu.sync_copy(rows.at[0], acc.at[pl.ds(r0 + i * CHUNK, CHUNK), :])
            return carry
        lax.fori_loop(0, rows_w // CHUNK, zcopy, 0)

        idx_wait(0)
        gather_start(0)
        idx_wait(1)
        gather_start(1)
        plsc.subcore_barrier()

        def step(k, carry):
            for j in range(RING):
                i = k * RING + j
                live = i < n_chunks

                @pl.when(live)
                def _():
                    gather_wait(j)
                    stash_idx(j)

                @pl.when(i + 3 < n_chunks)
                def _():
                    idx_start(i + 3, (j + 3) % RING)

                @pl.when(jnp.logical_and(i >= 2, i - 2 < n_chunks))
                def _():
                    scat_wait((j + 2) % RING)

                @pl.when(i + 2 < n_chunks)
                def _():
                    idx_wait((j + 2) % RING)
                    gather_start((j + 2) % RING)

                @pl.when(live)
                def _():
                    compute(j)
                    scat_start(j)
            return carry
        lax.fori_loop(0, n_iters // RING, step, 0)
        plsc.subcore_barrier()

        pltpu.sync_copy(acc.at[pl.ds(r0, rows_w), :],
                        out_hbm.at[c, pl.ds(r0, rows_w), :])

    return agg(features, eflat, wgt)


def _tc_combine_matmul_relu(partials, W, n_nodes):
    _, n_pad, d = partials.shape
    d_out = W.shape[1]
    blk = 1000
    assert n_nodes % blk == 0

    def body(p_ref, w_ref, o_ref):
        pp = p_ref[0] + p_ref[1]
        acc = jnp.dot(pp, w_ref[...], preferred_element_type=jnp.float32,
                      precision=lax.Precision.HIGHEST)
        o_ref[...] = jnp.maximum(acc, 0.0)

    return pl.pallas_call(
        body,
        grid=(n_nodes // blk,),
        in_specs=[
            pl.BlockSpec((NC, blk, d), lambda i: (0, i, 0)),
            pl.BlockSpec((d, d_out), lambda i: (0, 0)),
        ],
        out_specs=pl.BlockSpec((blk, d_out), lambda i: (i, 0)),
        out_shape=jax.ShapeDtypeStruct((n_nodes, d_out), jnp.float32),
    )(partials, W)


def kernel(features, edge_index, edge_weight, W):
    n_nodes = features.shape[0]
    n_pad = 10240
    eflat = edge_index.astype(jnp.int32).reshape(-1)
    wgt = edge_weight.astype(jnp.float32)
    partials = _sc_aggregate(features, eflat, wgt, n_pad)
    return _tc_combine_matmul_relu(partials, W, n_nodes)
